# baseline (speedup 1.0000x reference)
import functools

import jax
import jax.numpy as jnp
from jax import lax
from jax.experimental import pallas as pl
from jax.experimental.pallas import tpu as pltpu

M_BLK = 128
F_BLK = 256
_E = 8
_K = 2


def _moe_mlp_kernel(meta_ref, xs_ref, w1, w1s, w3, w3s, w2, w2s, out_ref,
                    r1, r1s, r3, r3s, r2, r2s, w1d, w3d, w2d, state, sems):
    f = pl.program_id(0)
    m = pl.program_id(1)
    nb = pl.num_programs(1)
    nf = pl.num_programs(0)
    n_active = meta_ref[nb]

    def copies(e_, f_, sl):
        fo = f_ * F_BLK
        return [
            pltpu.make_async_copy(w1.at[e_, pl.ds(fo, F_BLK), :],
                                  r1.at[sl], sems.at[sl]),
            pltpu.make_async_copy(w1s.at[e_, pl.ds(fo, F_BLK), :],
                                  r1s.at[sl], sems.at[sl]),
            pltpu.make_async_copy(w3.at[e_, pl.ds(fo, F_BLK), :],
                                  r3.at[sl], sems.at[sl]),
            pltpu.make_async_copy(w3s.at[e_, pl.ds(fo, F_BLK), :],
                                  r3s.at[sl], sems.at[sl]),
            pltpu.make_async_copy(w2.at[e_, :, pl.ds(fo, F_BLK)],
                                  r2.at[sl], sems.at[sl]),
            pltpu.make_async_copy(w2s.at[e_, :, pl.ds(fo, F_BLK)],
                                  r2s.at[sl], sems.at[sl]),
        ]

    @pl.when(jnp.logical_and(f == 0, m == 0))
    def _():
        state[0] = 0
        for c in copies(meta_ref[0], 0, 0):
            c.start()

    @pl.when(m < n_active)
    def _():
        be_m = meta_ref[m]
        prev = meta_ref[jnp.maximum(m - 1, 0)]
        is_start = jnp.logical_or(m == 0, be_m != prev)

        @pl.when(is_start)
        def _():
            r = state[0]
            slot = lax.rem(r, 2)
            nxt = lax.while_loop(
                lambda i: jnp.logical_and(i < n_active, meta_ref[i] == be_m),
                lambda i: i + 1, m + 1)
            in_sweep = nxt < n_active
            ne = jnp.where(in_sweep, meta_ref[jnp.minimum(nxt, nb - 1)],
                           meta_ref[0])
            nf_ = jnp.where(in_sweep, f, f + 1)

            @pl.when(jnp.logical_or(in_sweep, f + 1 < nf))
            def _():
                for c in copies(ne, nf_, lax.rem(r + 1, 2)):
                    c.start()

            for c in copies(0, 0, slot):
                c.wait()
            w1d[...] = (r1[slot] * r1s[slot]).astype(jnp.bfloat16)
            w3d[...] = (r3[slot] * r3s[slot]).astype(jnp.bfloat16)
            w2d[...] = (r2[slot] * r2s[slot]).astype(jnp.bfloat16)
            state[0] = r + 1

        x = xs_ref[pl.ds(m * M_BLK, M_BLK), :]
        dn = (((1,), (1,)), ((), ()))
        h1 = lax.dot_general(x, w1d[...], dn,
                             preferred_element_type=jnp.float32)
        h3 = lax.dot_general(x, w3d[...], dn,
                             preferred_element_type=jnp.float32)
        h = h1 * jax.nn.sigmoid(h1) * h3
        contrib = lax.dot_general(h.astype(jnp.bfloat16), w2d[...], dn,
                                  preferred_element_type=jnp.float32)
        sl_m = pl.ds(m * M_BLK, M_BLK)

        @pl.when(f == 0)
        def _():
            out_ref[sl_m, :] = contrib

        @pl.when(f != 0)
        def _():
            out_ref[sl_m, :] += contrib


@functools.partial(jax.jit, static_argnames=())
def kernel(hidden_states, gate_w, w1, w1_scale, w3, w3_scale, w2, w2_scale):
    b, s_len, d = hidden_states.shape
    e, f_dim, _ = w1.shape
    x = hidden_states.reshape(-1, d)
    t = x.shape[0]
    a = t * _K
    nb = (a + _E * (M_BLK - 1)) // M_BLK + 1
    p = nb * M_BLK
    nf = f_dim // F_BLK

    router_logits = x @ gate_w.T
    scores = jax.nn.sigmoid(router_logits)
    top_vals, top_idx = lax.top_k(scores, _K)
    routing_w = top_vals / jnp.sum(top_vals, axis=-1, keepdims=True)

    e_flat = top_idx.reshape(-1).astype(jnp.int32)
    oh = (e_flat[:, None] == jnp.arange(_E, dtype=jnp.int32)[None, :]
          ).astype(jnp.int32)
    csum = jnp.cumsum(oh, axis=0)
    counts = csum[-1]
    rank = jnp.take_along_axis(csum - oh, e_flat[:, None], axis=1)[:, 0]
    padded = ((counts + M_BLK - 1) // M_BLK) * M_BLK
    pad_cum = jnp.cumsum(padded)
    starts = pad_cum - padded
    dst = starts[e_flat] + rank
    tok_of_a = jnp.arange(a, dtype=jnp.int32) // _K
    src = jnp.zeros((p,), jnp.int32).at[dst].set(tok_of_a)
    block_expert = jnp.minimum(
        jnp.searchsorted(pad_cum, jnp.arange(nb, dtype=jnp.int32) * M_BLK,
                         side="right").astype(jnp.int32), _E - 1)
    n_active = (pad_cum[-1] // M_BLK).astype(jnp.int32)
    meta = jnp.concatenate([block_expert, n_active[None]])

    xs = x[src].astype(jnp.bfloat16)

    rows = pl.pallas_call(
        _moe_mlp_kernel,
        grid=(nf, nb),
        in_specs=[
            pl.BlockSpec(memory_space=pltpu.SMEM),
            pl.BlockSpec((p, d), lambda f, m: (0, 0)),
            pl.BlockSpec(memory_space=pltpu.HBM),
            pl.BlockSpec(memory_space=pltpu.HBM),
            pl.BlockSpec(memory_space=pltpu.HBM),
            pl.BlockSpec(memory_space=pltpu.HBM),
            pl.BlockSpec(memory_space=pltpu.HBM),
            pl.BlockSpec(memory_space=pltpu.HBM),
        ],
        out_specs=pl.BlockSpec((p, d), lambda f, m: (0, 0)),
        scratch_shapes=[
            pltpu.VMEM((2, F_BLK, d), jnp.float32),
            pltpu.VMEM((2, F_BLK, d), jnp.float32),
            pltpu.VMEM((2, F_BLK, d), jnp.float32),
            pltpu.VMEM((2, F_BLK, d), jnp.float32),
            pltpu.VMEM((2, d, F_BLK), jnp.float32),
            pltpu.VMEM((2, d, F_BLK), jnp.float32),
            pltpu.VMEM((F_BLK, d), jnp.bfloat16),
            pltpu.VMEM((F_BLK, d), jnp.bfloat16),
            pltpu.VMEM((d, F_BLK), jnp.bfloat16),
            pltpu.SMEM((2,), jnp.int32),
            pltpu.SemaphoreType.DMA((2,)),
        ],
        out_shape=jax.ShapeDtypeStruct((p, d), jnp.float32),
    )(meta, xs, w1, w1_scale, w3, w3_scale, w2, w2_scale)

    d0 = dst[0::2]
    d1 = dst[1::2]
    y = rows[d0] * routing_w[:, :1] + rows[d1] * routing_w[:, 1:]
    return y.reshape(b, s_len, d)


# speedup vs baseline: 1.2948x; 1.2948x over previous
import functools

import jax
import jax.numpy as jnp
from jax import lax
from jax.experimental import pallas as pl
from jax.experimental.pallas import tpu as pltpu
from jax.experimental.pallas import tpu_sc as plsc

M_BLK = 256
F_BLK = 256
_E = 8
_K = 2


def _sc_combine(rows, d0, d1, t, d):
    info = plsc.get_sparse_core_info()
    nw = info.num_cores * info.num_subcores
    per_w = t // nw
    chunk = min(per_w, 32)
    mesh = plsc.VectorSubcoreMesh(core_axis_name="c", subcore_axis_name="s")

    @functools.partial(
        pl.kernel, mesh=mesh,
        out_type=jax.ShapeDtypeStruct((t, d), jnp.float32),
        scratch_types=[
            pltpu.VMEM((chunk,), jnp.int32),
            pltpu.VMEM((chunk,), jnp.int32),
            pltpu.VMEM((chunk, d), jnp.float32),
            pltpu.VMEM((chunk, d), jnp.float32),
            pltpu.SemaphoreType.DMA,
        ],
    )
    def k(rows_hbm, d0_hbm, d1_hbm, y_hbm, idx0_v, idx1_v, buf0, buf1, sem):
        wid = lax.axis_index("s") * info.num_cores + lax.axis_index("c")
        base = wid * per_w
        for c in range(per_w // chunk):
            bc = base + c * chunk
            pltpu.sync_copy(d0_hbm.at[pl.ds(bc, chunk)], idx0_v)
            pltpu.sync_copy(d1_hbm.at[pl.ds(bc, chunk)], idx1_v)
            pltpu.async_copy(rows_hbm.at[idx0_v], buf0, sem).wait()
            pltpu.async_copy(rows_hbm.at[idx1_v], buf1, sem).wait()

            def body(tk, carry):
                for j in range(d // 16):
                    sl = pl.ds(j * 16, 16)
                    buf0[tk, sl] = buf0[tk, sl] + buf1[tk, sl]
                return carry

            lax.fori_loop(0, chunk, body, 0)
            pltpu.sync_copy(buf0, y_hbm.at[pl.ds(bc, chunk), :])

    return k(rows, d0, d1)


def _moe_mlp_kernel(meta_ref, xs_ref, ws_ref, w1, w1s, w3, w3s, w2, w2s,
                    out_ref, r1, r1s, r3, r3s, r2, r2s, w1d, w3d, w2d,
                    state, sems):
    f = pl.program_id(0)
    m = pl.program_id(1)
    nb = pl.num_programs(1)
    nf = pl.num_programs(0)
    n_active = meta_ref[nb]

    def copies(e_, f_, sl):
        fo = f_ * F_BLK
        return [
            pltpu.make_async_copy(w1.at[e_, pl.ds(fo, F_BLK), :],
                                  r1.at[sl], sems.at[sl]),
            pltpu.make_async_copy(w1s.at[e_, pl.ds(fo, F_BLK), :],
                                  r1s.at[sl], sems.at[sl]),
            pltpu.make_async_copy(w3.at[e_, pl.ds(fo, F_BLK), :],
                                  r3.at[sl], sems.at[sl]),
            pltpu.make_async_copy(w3s.at[e_, pl.ds(fo, F_BLK), :],
                                  r3s.at[sl], sems.at[sl]),
            pltpu.make_async_copy(w2.at[e_, :, pl.ds(fo, F_BLK)],
                                  r2.at[sl], sems.at[sl]),
            pltpu.make_async_copy(w2s.at[e_, :, pl.ds(fo, F_BLK)],
                                  r2s.at[sl], sems.at[sl]),
        ]

    @pl.when(jnp.logical_and(f == 0, m == 0))
    def _():
        state[0] = 0
        for c in copies(meta_ref[0], 0, 0):
            c.start()

    @pl.when(m < n_active)
    def _():
        be_m = meta_ref[m]
        prev = meta_ref[jnp.maximum(m - 1, 0)]
        is_start = jnp.logical_or(m == 0, be_m != prev)

        @pl.when(is_start)
        def _():
            r = state[0]
            slot = lax.rem(r, 2)
            nxt = lax.while_loop(
                lambda i: jnp.logical_and(i < n_active, meta_ref[i] == be_m),
                lambda i: i + 1, m + 1)
            in_sweep = nxt < n_active
            ne = jnp.where(in_sweep, meta_ref[jnp.minimum(nxt, nb - 1)],
                           meta_ref[0])
            nf_ = jnp.where(in_sweep, f, f + 1)

            @pl.when(jnp.logical_or(in_sweep, f + 1 < nf))
            def _():
                for c in copies(ne, nf_, lax.rem(r + 1, 2)):
                    c.start()

            for c in copies(0, 0, slot):
                c.wait()
            w1d[...] = (r1[slot] * r1s[slot]).astype(jnp.bfloat16)
            w3d[...] = (r3[slot] * r3s[slot]).astype(jnp.bfloat16)
            w2d[...] = (r2[slot] * r2s[slot]).astype(jnp.bfloat16)
            state[0] = r + 1

        x = xs_ref[pl.ds(m * M_BLK, M_BLK), :]
        dn = (((1,), (1,)), ((), ()))
        h1 = lax.dot_general(x, w1d[...], dn,
                             preferred_element_type=jnp.float32)
        h3 = lax.dot_general(x, w3d[...], dn,
                             preferred_element_type=jnp.float32)
        h = h1 * jax.nn.sigmoid(h1) * h3
        wv = ws_ref[pl.ds(m * M_BLK, M_BLK), :]
        contrib = lax.dot_general((h * wv).astype(jnp.bfloat16), w2d[...],
                                  dn, preferred_element_type=jnp.float32)
        sl_m = pl.ds(m * M_BLK, M_BLK)

        @pl.when(f == 0)
        def _():
            out_ref[sl_m, :] = contrib

        @pl.when(f != 0)
        def _():
            out_ref[sl_m, :] += contrib


@functools.partial(jax.jit, static_argnames=())
def kernel(hidden_states, gate_w, w1, w1_scale, w3, w3_scale, w2, w2_scale):
    b, s_len, d = hidden_states.shape
    e, f_dim, _ = w1.shape
    x = hidden_states.reshape(-1, d)
    t = x.shape[0]
    a = t * _K
    nb = (a + _E * (M_BLK - 1)) // M_BLK + 1
    p = nb * M_BLK
    nf = f_dim // F_BLK

    router_logits = x @ gate_w.T
    scores = jax.nn.sigmoid(router_logits)
    top_vals, top_idx = lax.top_k(scores, _K)
    routing_w = top_vals / jnp.sum(top_vals, axis=-1, keepdims=True)

    e_flat = top_idx.reshape(-1).astype(jnp.int32)
    oh = (e_flat[:, None] == jnp.arange(_E, dtype=jnp.int32)[None, :]
          ).astype(jnp.int32)
    csum = jnp.cumsum(oh, axis=0)
    counts = csum[-1]
    rank = jnp.take_along_axis(csum - oh, e_flat[:, None], axis=1)[:, 0]
    padded = ((counts + M_BLK - 1) // M_BLK) * M_BLK
    pad_cum = jnp.cumsum(padded)
    starts = pad_cum - padded
    dst = starts[e_flat] + rank
    tok_of_a = jnp.arange(a, dtype=jnp.int32) // _K
    src = jnp.zeros((p,), jnp.int32).at[dst].set(tok_of_a)
    block_expert = jnp.minimum(
        jnp.searchsorted(pad_cum, jnp.arange(nb, dtype=jnp.int32) * M_BLK,
                         side="right").astype(jnp.int32), _E - 1)
    n_active = (pad_cum[-1] // M_BLK).astype(jnp.int32)
    meta = jnp.concatenate([block_expert, n_active[None]])

    xs = x[src].astype(jnp.bfloat16)
    ws = jnp.zeros((p, 1), jnp.float32).at[dst, 0].set(routing_w.reshape(-1))

    rows = pl.pallas_call(
        _moe_mlp_kernel,
        grid=(nf, nb),
        in_specs=[
            pl.BlockSpec(memory_space=pltpu.SMEM),
            pl.BlockSpec((p, d), lambda f, m: (0, 0)),
            pl.BlockSpec((p, 1), lambda f, m: (0, 0)),
            pl.BlockSpec(memory_space=pltpu.HBM),
            pl.BlockSpec(memory_space=pltpu.HBM),
            pl.BlockSpec(memory_space=pltpu.HBM),
            pl.BlockSpec(memory_space=pltpu.HBM),
            pl.BlockSpec(memory_space=pltpu.HBM),
            pl.BlockSpec(memory_space=pltpu.HBM),
        ],
        out_specs=pl.BlockSpec((p, d), lambda f, m: (0, 0)),
        scratch_shapes=[
            pltpu.VMEM((2, F_BLK, d), jnp.float32),
            pltpu.VMEM((2, F_BLK, d), jnp.float32),
            pltpu.VMEM((2, F_BLK, d), jnp.float32),
            pltpu.VMEM((2, F_BLK, d), jnp.float32),
            pltpu.VMEM((2, d, F_BLK), jnp.float32),
            pltpu.VMEM((2, d, F_BLK), jnp.float32),
            pltpu.VMEM((F_BLK, d), jnp.bfloat16),
            pltpu.VMEM((F_BLK, d), jnp.bfloat16),
            pltpu.VMEM((d, F_BLK), jnp.bfloat16),
            pltpu.SMEM((2,), jnp.int32),
            pltpu.SemaphoreType.DMA((2,)),
        ],
        out_shape=jax.ShapeDtypeStruct((p, d), jnp.float32),
    )(meta, xs, ws, w1, w1_scale, w3, w3_scale, w2, w2_scale)

    d0 = dst[0::2]
    d1 = dst[1::2]
    y = _sc_combine(rows, d0, d1, t, d)
    return y.reshape(b, s_len, d)
